# Initial kernel scaffold; baseline (speedup 1.0000x reference)
import jax
import jax.numpy as jnp
from jax.experimental import pallas as pl


def kernel(x, weight, bias):
    raise NotImplementedError("write your pallas kernel here")


# speedup vs baseline: 3.0527x; 3.0527x over previous
import jax
import jax.numpy as jnp
from jax.experimental import pallas as pl
from jax.experimental.pallas import tpu as pltpu


def _pick_tile(dim, candidates):
    for c in candidates:
        if dim >= c and dim % c == 0:
            return c
    return dim


def _addmm_body(x_ref, w_ref, b_ref, o_ref):
    xb = x_ref[...].astype(jnp.bfloat16)
    acc = jnp.dot(xb, w_ref[...], preferred_element_type=jnp.float32)
    o_ref[...] = (acc + b_ref[...].astype(jnp.float32)).astype(o_ref.dtype)


def _addmm(x2d, w_bf16, bias):
    M, nx = x2d.shape
    nf = w_bf16.shape[1]
    tm = _pick_tile(M, (1024, 512, 256, 8))
    tn = _pick_tile(nf, (1024, 512, 256, 128))
    b2d = bias.reshape(1, nf)
    cost = pl.CostEstimate(
        flops=2 * M * nx * nf,
        transcendentals=0,
        bytes_accessed=(M * nx * 4 + (M // tm) * nx * nf * 2
                        + nf * 4 + M * nf * 4))
    return pl.pallas_call(
        _addmm_body,
        out_shape=jax.ShapeDtypeStruct((M, nf), x2d.dtype),
        grid=(M // tm, nf // tn),
        in_specs=[
            pl.BlockSpec((tm, nx), lambda i, j: (i, 0)),
            pl.BlockSpec((nx, tn), lambda i, j: (0, j)),
            pl.BlockSpec((1, tn), lambda i, j: (0, j)),
        ],
        out_specs=pl.BlockSpec((tm, tn), lambda i, j: (i, j)),
        compiler_params=pltpu.CompilerParams(
            dimension_semantics=("parallel", "parallel")),
        cost_estimate=cost,
    )(x2d, w_bf16, b2d)


def kernel(x, weight, bias):
    nf = weight.shape[1]
    size_out = x.shape[:-1] + (nf,)
    x2d = x.reshape(-1, x.shape[-1])
    y = _addmm(x2d, weight.astype(jnp.bfloat16), bias)
    return y.reshape(size_out)
